# baseline (device time: 29892 ns/iter reference)
import jax
import jax.numpy as jnp
from jax import lax
from jax.experimental import pallas as pl
from jax.experimental.pallas import tpu as pltpu

T = 1024
D = 2048
V_LOCAL = 16384
V_SUB = 2048
N_CHUNKS = 4
CW = V_SUB // N_CHUNKS
N_DEV = 16


def _body(x_ref, w_ref, l_ref, out_ref, wv, stbuf, allrecv,
          dma_sems, ssems, rsems):
    my_x = lax.axis_index("x")
    my_y = lax.axis_index("y")
    my_z = lax.axis_index("z")
    r = my_x * 4 + my_z
    c0 = r * V_SUB
    my_g = my_x * 8 + my_y * 4 + my_z

    def peer(o):
        pr = (my_g + o) % N_DEV
        return (pr // 8, (pr // 4) % 2, pr % 4)

    barrier = pltpu.get_barrier_semaphore()
    for o in range(1, N_DEV):
        pl.semaphore_signal(barrier, inc=1, device_id=peer(o),
                            device_id_type=pl.DeviceIdType.MESH)

    cps = []
    for h in range(N_CHUNKS):
        cp = pltpu.make_async_copy(
            w_ref.at[:, pl.ds(c0 + h * CW, CW)],
            wv.at[:, pl.ds(h * CW, CW)],
            dma_sems.at[h],
        )
        cp.start()
        cps.append(cp)

    col_base = my_y * V_LOCAL + c0
    ones_row = jnp.ones((1, CW), jnp.float32)
    red_dims = (((1,), (1,)), ((), ()))
    s_row = jnp.zeros((1, T), jnp.float32)
    lle_row = jnp.zeros((1, T), jnp.float32)

    def mm(h):
        return lax.dot_general(
            x_ref[...], wv[:, h * CW:(h + 1) * CW],
            (((1,), (0,)), ((), ())),
            preferred_element_type=jnp.float32,
            precision=lax.Precision.DEFAULT,
        )

    def vpu(h, logits, s_row, lle_row):
        e = jnp.exp(logits)
        cols = lax.broadcasted_iota(jnp.int32, (T, CW), 1) + (
            col_base + h * CW
        )
        masked = jnp.where(cols == l_ref[...], e, 0.0)
        s_row += lax.dot_general(
            ones_row, e, red_dims,
            preferred_element_type=jnp.float32,
            precision=lax.Precision.DEFAULT,
        )
        lle_row += lax.dot_general(
            ones_row, masked, red_dims,
            preferred_element_type=jnp.float32,
            precision=lax.Precision.DEFAULT,
        )
        return s_row, lle_row

    cps[0].wait()
    l_prev = mm(0)
    for h in range(1, N_CHUNKS):
        cps[h].wait()
        l_cur = mm(h)
        s_row, lle_row = vpu(h - 1, l_prev, s_row, lle_row)
        l_prev = l_cur
    s_row, lle_row = vpu(N_CHUNKS - 1, l_prev, s_row, lle_row)

    stbuf[0:8, :] = s_row.reshape(8, 128)
    stbuf[8:16, :] = lle_row.reshape(8, 128)

    pl.semaphore_wait(barrier, N_DEV - 1)

    copies = []
    for o in range(1, N_DEV):
        c = pltpu.make_async_remote_copy(
            src_ref=stbuf, dst_ref=allrecv.at[N_DEV - 1 - o],
            send_sem=ssems.at[o - 1], recv_sem=rsems.at[N_DEV - 1 - o],
            device_id=peer(o), device_id_type=pl.DeviceIdType.MESH,
        )
        c.start()
        copies.append(c)
    for c in copies:
        c.wait()

    total = stbuf[...]
    for k in range(N_DEV - 1):
        total += allrecv[k]

    out_ref[...] = jnp.log(total[0:8, :]) - jnp.log(total[8:16, :])


def kernel(x, W, labels):
    labels2d = labels.reshape(T, 1)

    nll = pl.pallas_call(
        _body,
        in_specs=[
            pl.BlockSpec(memory_space=pltpu.VMEM),
            pl.BlockSpec(memory_space=pl.ANY),
            pl.BlockSpec(memory_space=pltpu.VMEM),
        ],
        out_specs=pl.BlockSpec(memory_space=pltpu.VMEM),
        out_shape=jax.ShapeDtypeStruct((8, 128), jnp.float32),
        scratch_shapes=[
            pltpu.VMEM((D, V_SUB), jnp.float32),
            pltpu.VMEM((16, 128), jnp.float32),
            pltpu.VMEM((N_DEV - 1, 16, 128), jnp.float32),
            pltpu.SemaphoreType.DMA((N_CHUNKS,)),
            pltpu.SemaphoreType.DMA((N_DEV - 1,)),
            pltpu.SemaphoreType.DMA((N_DEV - 1,)),
        ],
        compiler_params=pltpu.CompilerParams(
            collective_id=0,
            vmem_limit_bytes=100 * 1024 * 1024,
        ),
    )(x, W, labels2d)

    return nll.reshape(T)
